# baseline (speedup 1.0000x reference)
import functools

import jax
import jax.numpy as jnp
from jax import lax
from jax.experimental import pallas as pl
from jax.experimental.pallas import tpu as pltpu
from jax.experimental.pallas import tpu_sc as plsc

F32 = jnp.float32
I32 = jnp.int32

NN = 10000
EE = 320000
DIN = 128
HID = 64
NG = 64
NMETA = 27

NC, NS = 2, 16
NW = NC * NS
CH = 128
KCH = 80
CHG = 4
NCHK = KCH // CHG
EPAD = NW * KCH * CH
NSP = 10016
RPT = NSP // NS
DW = 8
NBUF = 2

_mesh = plsc.VectorSubcoreMesh(
    core_axis_name="c", subcore_axis_name="s", num_cores=NC, num_subcores=NS)


@functools.partial(
    pl.kernel,
    out_type=(jax.ShapeDtypeStruct((NSP, DW), F32),
              jax.ShapeDtypeStruct((NSP, DW), F32)),
    mesh=_mesh,
    scratch_types=[
        pltpu.VMEM((KCH * CH,), I32),
        pltpu.VMEM((CH, DW), F32),
        pltpu.VMEM_SHARED((NSP, DW), F32),
        pltpu.SemaphoreType.DMA,
    ],
    compiler_params=pltpu.CompilerParams(use_tc_tiling_on_sc=False),
)
def _sc_degree(dst_hbm, zeros_hbm, ones_hbm, deg0_hbm, deg1_hbm,
               dstv, onesv, deg_sh, sem):
    cid = lax.axis_index("c")
    sid = lax.axis_index("s")
    wid = cid * NS + sid
    pltpu.sync_copy(zeros_hbm.at[pl.ds(sid * RPT, RPT)],
                    deg_sh.at[pl.ds(sid * RPT, RPT)])
    pltpu.sync_copy(ones_hbm, onesv)
    pltpu.async_copy(dst_hbm.at[wid], dstv, sem).wait()
    plsc.subcore_barrier()

    def body(j, carry):
        pltpu.async_copy(onesv, deg_sh.at[dstv.at[pl.ds(j * CH, CH)]],
                         sem, add=True)

        @pl.when(j >= 8)
        def _():
            pltpu.make_async_copy(onesv, deg_sh.at[dstv.at[pl.ds(0, CH)]],
                                  sem).wait()

        return carry

    lax.fori_loop(0, KCH, body, 0)
    for _ in range(8):
        pltpu.make_async_copy(onesv, deg_sh.at[dstv.at[pl.ds(0, CH)]],
                              sem).wait()
    plsc.subcore_barrier()

    @pl.when(cid == 0)
    def _():
        pltpu.sync_copy(deg_sh.at[pl.ds(sid * RPT, RPT)],
                        deg0_hbm.at[pl.ds(sid * RPT, RPT)])

    @pl.when(cid == 1)
    def _():
        pltpu.sync_copy(deg_sh.at[pl.ds(sid * RPT, RPT)],
                        deg1_hbm.at[pl.ds(sid * RPT, RPT)])


@functools.partial(
    pl.kernel,
    out_type=(jax.ShapeDtypeStruct((NSP, HID), F32),
              jax.ShapeDtypeStruct((NSP, HID), F32)),
    mesh=_mesh,
    scratch_types=[
        pltpu.VMEM((KCH * CH,), I32),
        pltpu.VMEM((KCH * CH,), I32),
        [pltpu.VMEM((CHG * CH, HID), F32) for _ in range(NBUF)],
        pltpu.VMEM_SHARED((NSP, HID), F32),
        [pltpu.SemaphoreType.DMA for _ in range(NBUF)],
        [pltpu.SemaphoreType.DMA for _ in range(NBUF)],
        pltpu.SemaphoreType.DMA,
    ],
    compiler_params=pltpu.CompilerParams(use_tc_tiling_on_sc=False),
)
def _sc_edge_pass(src_hbm, dst_hbm, u_hbm, zeros_hbm, s0_hbm, s1_hbm,
                  srcv, dstv, bufs, s_sh, gsems, ssems, semi):
    cid = lax.axis_index("c")
    sid = lax.axis_index("s")
    wid = cid * NS + sid
    pltpu.async_copy(src_hbm.at[wid], srcv, semi).wait()
    for b in range(NBUF):
        pltpu.async_copy(u_hbm.at[srcv.at[pl.ds(b * CHG * CH, CHG * CH)]], bufs[b],
                         gsems[b])
    pltpu.async_copy(dst_hbm.at[wid], dstv, semi).wait()
    pltpu.sync_copy(zeros_hbm.at[pl.ds(sid * RPT, RPT)],
                    s_sh.at[pl.ds(sid * RPT, RPT)])
    plsc.subcore_barrier()


    def body(i, carry):
        j0 = NBUF * i
        for b in range(NBUF):
            j = j0 + b

            @pl.when(j < NCHK)
            def _(b=b, j=j):
                pltpu.make_async_copy(
                    u_hbm.at[srcv.at[pl.ds(j * CHG * CH, CHG * CH)]], bufs[b],
                    gsems[b]).wait()
                pltpu.async_copy(bufs[b], s_sh.at[dstv.at[pl.ds(j * CHG * CH, CHG * CH)]],
                                 ssems[b], add=True)

        for b in range(NBUF):
            j = j0 + b

            @pl.when(j + NBUF < NCHK)
            def _(b=b, j=j):
                pltpu.make_async_copy(bufs[b], s_sh.at[dstv.at[pl.ds(0, CHG * CH)]],
                                      ssems[b]).wait()
                pltpu.async_copy(
                    u_hbm.at[srcv.at[pl.ds((j + NBUF) * CHG * CH, CHG * CH)]], bufs[b],
                    gsems[b])

        return carry

    lax.fori_loop(0, (NCHK + NBUF - 1) // NBUF, body, 0)
    for b in range(NBUF):
        pltpu.make_async_copy(bufs[b], s_sh.at[dstv.at[pl.ds(0, CHG * CH)]],
                              ssems[b]).wait()
    plsc.subcore_barrier()

    @pl.when(cid == 0)
    def _():
        pltpu.sync_copy(s_sh.at[pl.ds(sid * RPT, RPT)],
                        s0_hbm.at[pl.ds(sid * RPT, RPT)])

    @pl.when(cid == 1)
    def _():
        pltpu.sync_copy(s_sh.at[pl.ds(sid * RPT, RPT)],
                        s1_hbm.at[pl.ds(sid * RPT, RPT)])


_RB = 400
_NB = NN // _RB


def _tc_b_body(x_ref, d0_ref, d1_ref, w1_ref, h_ref, u_ref, dis_ref):
    deg = d0_ref[:, 0:1] + d1_ref[:, 0:1] + 1.0
    dis = lax.rsqrt(deg)
    h = jnp.dot(x_ref[...], w1_ref[...], preferred_element_type=F32)
    h_ref[...] = h
    u_ref[...] = h * dis
    dis_ref[...] = dis


def _tc_stage_b(x, deg0, deg1, w1):
    return pl.pallas_call(
        _tc_b_body,
        grid=(_NB,),
        in_specs=[
            pl.BlockSpec((_RB, DIN), lambda i: (i, 0)),
            pl.BlockSpec((_RB, DW), lambda i: (i, 0)),
            pl.BlockSpec((_RB, DW), lambda i: (i, 0)),
            pl.BlockSpec((DIN, HID), lambda i: (0, 0)),
        ],
        out_specs=[
            pl.BlockSpec((_RB, HID), lambda i: (i, 0)),
            pl.BlockSpec((_RB, HID), lambda i: (i, 0)),
            pl.BlockSpec((_RB, 1), lambda i: (i, 0)),
        ],
        out_shape=[
            jax.ShapeDtypeStruct((NN, HID), F32),
            jax.ShapeDtypeStruct((NN, HID), F32),
            jax.ShapeDtypeStruct((NN, 1), F32),
        ],
    )(x, deg0, deg1, w1)


def _tc_d_body(s0_ref, s1_ref, h1_ref, dis_ref, b1_ref, w2_ref,
               h2_ref, u2_ref):
    dis = dis_ref[...]
    a = dis * (s0_ref[...] + s1_ref[...]) + dis * dis * h1_ref[...] \
        + b1_ref[...]
    a = jnp.maximum(a, 0.0)
    h2 = jnp.dot(a, w2_ref[...], preferred_element_type=F32)
    h2_ref[...] = h2
    u2_ref[...] = h2 * dis


def _tc_stage_d(s0, s1, h1, dis, b1, w2):
    return pl.pallas_call(
        _tc_d_body,
        grid=(_NB,),
        in_specs=[
            pl.BlockSpec((_RB, HID), lambda i: (i, 0)),
            pl.BlockSpec((_RB, HID), lambda i: (i, 0)),
            pl.BlockSpec((_RB, HID), lambda i: (i, 0)),
            pl.BlockSpec((_RB, 1), lambda i: (i, 0)),
            pl.BlockSpec((1, HID), lambda i: (0, 0)),
            pl.BlockSpec((HID, HID), lambda i: (0, 0)),
        ],
        out_specs=[
            pl.BlockSpec((_RB, HID), lambda i: (i, 0)),
            pl.BlockSpec((_RB, HID), lambda i: (i, 0)),
        ],
        out_shape=[
            jax.ShapeDtypeStruct((NN, HID), F32),
            jax.ShapeDtypeStruct((NN, HID), F32),
        ],
    )(s0, s1, h1, dis, b1, w2)


def _tc_f_body(s0_ref, s1_ref, h2_ref, dis_ref, b2_ref, batch_ref, meta_ref,
               wh1a_ref, wh1b_ref, bh1_ref, wh2_ref, bh2_ref,
               out_ref, acc_e, acc_c):
    i = pl.program_id(0)

    @pl.when(i == 0)
    def _():
        acc_e[...] = jnp.zeros((NG, HID), F32)
        acc_c[...] = jnp.zeros((NG, HID), F32)

    dis = dis_ref[...]
    a = dis * (s0_ref[...] + s1_ref[...]) + dis * dis * h2_ref[...] \
        + b2_ref[...]
    a = jnp.maximum(a, 0.0)
    gids = lax.broadcasted_iota(I32, (_RB, NG), 1)
    onehot = (batch_ref[...] == gids).astype(F32)
    dn = (((0,), (0,)), ((), ()))
    acc_e[...] += lax.dot_general(onehot, a, dn,
                                  preferred_element_type=F32)
    acc_c[...] += lax.dot_general(onehot, jnp.ones((_RB, HID), F32), dn,
                                  preferred_element_type=F32)

    @pl.when(i == _NB - 1)
    def _():
        emb = acc_e[...] / jnp.maximum(acc_c[...], 1.0)
        z = jnp.dot(emb, wh1a_ref[...], preferred_element_type=F32) \
            + jnp.dot(meta_ref[...], wh1b_ref[...],
                      preferred_element_type=F32) + bh1_ref[...]
        z = jnp.maximum(z, 0.0)
        out_ref[...] = jnp.dot(z, wh2_ref[...],
                               preferred_element_type=F32) + bh2_ref[...]


def _tc_stage_f(s0, s1, h2, dis, b2, batch2, meta, wh1a, wh1b, bh1, wh2, bh2):
    return pl.pallas_call(
        _tc_f_body,
        grid=(_NB,),
        in_specs=[
            pl.BlockSpec((_RB, HID), lambda i: (i, 0)),
            pl.BlockSpec((_RB, HID), lambda i: (i, 0)),
            pl.BlockSpec((_RB, HID), lambda i: (i, 0)),
            pl.BlockSpec((_RB, 1), lambda i: (i, 0)),
            pl.BlockSpec((1, HID), lambda i: (0, 0)),
            pl.BlockSpec((_RB, 1), lambda i: (i, 0)),
            pl.BlockSpec((NG, NMETA), lambda i: (0, 0)),
            pl.BlockSpec((HID, HID), lambda i: (0, 0)),
            pl.BlockSpec((NMETA, HID), lambda i: (0, 0)),
            pl.BlockSpec((1, HID), lambda i: (0, 0)),
            pl.BlockSpec((HID, 1), lambda i: (0, 0)),
            pl.BlockSpec((1, 1), lambda i: (0, 0)),
        ],
        out_specs=pl.BlockSpec((NG, 1), lambda i: (0, 0)),
        out_shape=jax.ShapeDtypeStruct((NG, 1), F32),
        scratch_shapes=[
            pltpu.VMEM((NG, HID), F32),
            pltpu.VMEM((NG, HID), F32),
        ],
    )(s0, s1, h2, dis, b2, batch2, meta, wh1a, wh1b, bh1, wh2, bh2)


def kernel(x, edge_index, batch, metadata, W1, b1, W2, b2, Wh1, bh1, Wh2, bh2):
    npad = EPAD - EE
    pad_src = (jnp.arange(npad, dtype=I32) * 131) % NN
    pad_dst = NN + (jnp.arange(npad, dtype=I32) % 16)
    src3 = jnp.concatenate([edge_index[0], pad_src]).reshape(NW, KCH * CH)
    dst3 = jnp.concatenate([edge_index[1], pad_dst]).reshape(NW, KCH * CH)

    zeros_h = jnp.zeros((NSP, HID), F32)
    zeros_d = jnp.zeros((NSP, DW), F32)
    ones_d = jnp.ones((CH, DW), F32)

    deg0, deg1 = _sc_degree(dst3, zeros_d, ones_d)
    h1, u1, dis = _tc_stage_b(x, deg0, deg1, W1)
    s10, s11 = _sc_edge_pass(src3, dst3, u1, zeros_h)
    h2, u2 = _tc_stage_d(s10, s11, h1, dis, b1.reshape(1, HID), W2)
    s20, s21 = _sc_edge_pass(src3, dst3, u2, zeros_h)
    out = _tc_stage_f(s20, s21, h2, dis, b2.reshape(1, HID),
                      batch.reshape(NN, 1), metadata,
                      Wh1[:HID], Wh1[HID:], bh1.reshape(1, HID),
                      Wh2, bh2.reshape(1, 1))
    return out


# speedup vs baseline: 1.1078x; 1.1078x over previous
import functools

import jax
import jax.numpy as jnp
from jax import lax
from jax.experimental import pallas as pl
from jax.experimental.pallas import tpu as pltpu
from jax.experimental.pallas import tpu_sc as plsc

F32 = jnp.float32
I32 = jnp.int32

NN = 10000
EE = 320000
DIN = 128
HID = 64
NG = 64
NMETA = 27

NC, NS = 2, 16
NW = NC * NS
CH = 128
KCH = 80
CHG = 2
NCHK = KCH // CHG
EPAD = NW * KCH * CH
NSP = 10016
RPT = NSP // NS
DW = 8
NBUF = 4

_mesh = plsc.VectorSubcoreMesh(
    core_axis_name="c", subcore_axis_name="s", num_cores=NC, num_subcores=NS)


@functools.partial(
    pl.kernel,
    out_type=(jax.ShapeDtypeStruct((NSP, DW), F32),
              jax.ShapeDtypeStruct((NSP, DW), F32)),
    mesh=_mesh,
    scratch_types=[
        pltpu.VMEM((KCH * CH,), I32),
        pltpu.VMEM((CH, DW), F32),
        pltpu.VMEM_SHARED((NSP, DW), F32),
        pltpu.SemaphoreType.DMA,
    ],
    compiler_params=pltpu.CompilerParams(use_tc_tiling_on_sc=False),
)
def _sc_degree(dst_hbm, zeros_hbm, ones_hbm, deg0_hbm, deg1_hbm,
               dstv, onesv, deg_sh, sem):
    cid = lax.axis_index("c")
    sid = lax.axis_index("s")
    wid = cid * NS + sid
    pltpu.sync_copy(zeros_hbm.at[pl.ds(sid * RPT, RPT)],
                    deg_sh.at[pl.ds(sid * RPT, RPT)])
    pltpu.sync_copy(ones_hbm, onesv)
    pltpu.async_copy(dst_hbm.at[wid], dstv, sem).wait()
    plsc.subcore_barrier()

    def body(j, carry):
        pltpu.async_copy(onesv, deg_sh.at[dstv.at[pl.ds(j * CH, CH)]],
                         sem, add=True)

        @pl.when(j >= 8)
        def _():
            pltpu.make_async_copy(onesv, deg_sh.at[dstv.at[pl.ds(0, CH)]],
                                  sem).wait()

        return carry

    lax.fori_loop(0, KCH, body, 0)
    for _ in range(8):
        pltpu.make_async_copy(onesv, deg_sh.at[dstv.at[pl.ds(0, CH)]],
                              sem).wait()
    plsc.subcore_barrier()

    @pl.when(cid == 0)
    def _():
        pltpu.sync_copy(deg_sh.at[pl.ds(sid * RPT, RPT)],
                        deg0_hbm.at[pl.ds(sid * RPT, RPT)])

    @pl.when(cid == 1)
    def _():
        pltpu.sync_copy(deg_sh.at[pl.ds(sid * RPT, RPT)],
                        deg1_hbm.at[pl.ds(sid * RPT, RPT)])


@functools.partial(
    pl.kernel,
    out_type=(jax.ShapeDtypeStruct((NSP, HID), F32),
              jax.ShapeDtypeStruct((NSP, HID), F32)),
    mesh=_mesh,
    scratch_types=[
        pltpu.VMEM((KCH * CH,), I32),
        pltpu.VMEM((KCH * CH,), I32),
        [pltpu.VMEM((CHG * CH, HID), F32) for _ in range(NBUF)],
        pltpu.VMEM_SHARED((NSP, HID), F32),
        [pltpu.SemaphoreType.DMA for _ in range(NBUF)],
        [pltpu.SemaphoreType.DMA for _ in range(NBUF)],
        pltpu.SemaphoreType.DMA,
    ],
    compiler_params=pltpu.CompilerParams(use_tc_tiling_on_sc=False),
)
def _sc_edge_pass(src_hbm, dst_hbm, u_hbm, zeros_hbm, s0_hbm, s1_hbm,
                  srcv, dstv, bufs, s_sh, gsems, ssems, semi):
    cid = lax.axis_index("c")
    sid = lax.axis_index("s")
    wid = cid * NS + sid
    pltpu.async_copy(src_hbm.at[wid], srcv, semi).wait()
    for b in range(NBUF):
        pltpu.async_copy(u_hbm.at[srcv.at[pl.ds(b * CHG * CH, CHG * CH)]], bufs[b],
                         gsems[b])
    pltpu.async_copy(dst_hbm.at[wid], dstv, semi).wait()
    pltpu.sync_copy(zeros_hbm.at[pl.ds(sid * RPT, RPT)],
                    s_sh.at[pl.ds(sid * RPT, RPT)])
    plsc.subcore_barrier()


    def body(i, carry):
        j0 = NBUF * i
        for b in range(NBUF):
            j = j0 + b

            @pl.when(j < NCHK)
            def _(b=b, j=j):
                pltpu.make_async_copy(
                    u_hbm.at[srcv.at[pl.ds(j * CHG * CH, CHG * CH)]], bufs[b],
                    gsems[b]).wait()
                pltpu.async_copy(bufs[b], s_sh.at[dstv.at[pl.ds(j * CHG * CH, CHG * CH)]],
                                 ssems[b], add=True)

        for b in range(NBUF):
            j = j0 + b

            @pl.when(j + NBUF < NCHK)
            def _(b=b, j=j):
                pltpu.make_async_copy(bufs[b], s_sh.at[dstv.at[pl.ds(0, CHG * CH)]],
                                      ssems[b]).wait()
                pltpu.async_copy(
                    u_hbm.at[srcv.at[pl.ds((j + NBUF) * CHG * CH, CHG * CH)]], bufs[b],
                    gsems[b])

        return carry

    lax.fori_loop(0, (NCHK + NBUF - 1) // NBUF, body, 0)
    for b in range(NBUF):
        pltpu.make_async_copy(bufs[b], s_sh.at[dstv.at[pl.ds(0, CHG * CH)]],
                              ssems[b]).wait()
    plsc.subcore_barrier()

    @pl.when(cid == 0)
    def _():
        pltpu.sync_copy(s_sh.at[pl.ds(sid * RPT, RPT)],
                        s0_hbm.at[pl.ds(sid * RPT, RPT)])

    @pl.when(cid == 1)
    def _():
        pltpu.sync_copy(s_sh.at[pl.ds(sid * RPT, RPT)],
                        s1_hbm.at[pl.ds(sid * RPT, RPT)])


_RB = 400
_NB = NN // _RB


def _tc_b_body(x_ref, d0_ref, d1_ref, w1_ref, h_ref, u_ref, dis_ref):
    deg = d0_ref[:, 0:1] + d1_ref[:, 0:1] + 1.0
    dis = lax.rsqrt(deg)
    h = jnp.dot(x_ref[...], w1_ref[...], preferred_element_type=F32)
    h_ref[...] = h
    u_ref[...] = h * dis
    dis_ref[...] = dis


def _tc_stage_b(x, deg0, deg1, w1):
    return pl.pallas_call(
        _tc_b_body,
        grid=(_NB,),
        in_specs=[
            pl.BlockSpec((_RB, DIN), lambda i: (i, 0)),
            pl.BlockSpec((_RB, DW), lambda i: (i, 0)),
            pl.BlockSpec((_RB, DW), lambda i: (i, 0)),
            pl.BlockSpec((DIN, HID), lambda i: (0, 0)),
        ],
        out_specs=[
            pl.BlockSpec((_RB, HID), lambda i: (i, 0)),
            pl.BlockSpec((_RB, HID), lambda i: (i, 0)),
            pl.BlockSpec((_RB, 1), lambda i: (i, 0)),
        ],
        out_shape=[
            jax.ShapeDtypeStruct((NN, HID), F32),
            jax.ShapeDtypeStruct((NN, HID), F32),
            jax.ShapeDtypeStruct((NN, 1), F32),
        ],
    )(x, deg0, deg1, w1)


def _tc_d_body(s0_ref, s1_ref, h1_ref, dis_ref, b1_ref, w2_ref,
               h2_ref, u2_ref):
    dis = dis_ref[...]
    a = dis * (s0_ref[...] + s1_ref[...]) + dis * dis * h1_ref[...] \
        + b1_ref[...]
    a = jnp.maximum(a, 0.0)
    h2 = jnp.dot(a, w2_ref[...], preferred_element_type=F32)
    h2_ref[...] = h2
    u2_ref[...] = h2 * dis


def _tc_stage_d(s0, s1, h1, dis, b1, w2):
    return pl.pallas_call(
        _tc_d_body,
        grid=(_NB,),
        in_specs=[
            pl.BlockSpec((_RB, HID), lambda i: (i, 0)),
            pl.BlockSpec((_RB, HID), lambda i: (i, 0)),
            pl.BlockSpec((_RB, HID), lambda i: (i, 0)),
            pl.BlockSpec((_RB, 1), lambda i: (i, 0)),
            pl.BlockSpec((1, HID), lambda i: (0, 0)),
            pl.BlockSpec((HID, HID), lambda i: (0, 0)),
        ],
        out_specs=[
            pl.BlockSpec((_RB, HID), lambda i: (i, 0)),
            pl.BlockSpec((_RB, HID), lambda i: (i, 0)),
        ],
        out_shape=[
            jax.ShapeDtypeStruct((NN, HID), F32),
            jax.ShapeDtypeStruct((NN, HID), F32),
        ],
    )(s0, s1, h1, dis, b1, w2)


def _tc_f_body(s0_ref, s1_ref, h2_ref, dis_ref, b2_ref, batch_ref, meta_ref,
               wh1a_ref, wh1b_ref, bh1_ref, wh2_ref, bh2_ref,
               out_ref, acc_e, acc_c):
    i = pl.program_id(0)

    @pl.when(i == 0)
    def _():
        acc_e[...] = jnp.zeros((NG, HID), F32)
        acc_c[...] = jnp.zeros((NG, HID), F32)

    dis = dis_ref[...]
    a = dis * (s0_ref[...] + s1_ref[...]) + dis * dis * h2_ref[...] \
        + b2_ref[...]
    a = jnp.maximum(a, 0.0)
    gids = lax.broadcasted_iota(I32, (_RB, NG), 1)
    onehot = (batch_ref[...] == gids).astype(F32)
    dn = (((0,), (0,)), ((), ()))
    acc_e[...] += lax.dot_general(onehot, a, dn,
                                  preferred_element_type=F32)
    acc_c[...] += lax.dot_general(onehot, jnp.ones((_RB, HID), F32), dn,
                                  preferred_element_type=F32)

    @pl.when(i == _NB - 1)
    def _():
        emb = acc_e[...] / jnp.maximum(acc_c[...], 1.0)
        z = jnp.dot(emb, wh1a_ref[...], preferred_element_type=F32) \
            + jnp.dot(meta_ref[...], wh1b_ref[...],
                      preferred_element_type=F32) + bh1_ref[...]
        z = jnp.maximum(z, 0.0)
        out_ref[...] = jnp.dot(z, wh2_ref[...],
                               preferred_element_type=F32) + bh2_ref[...]


def _tc_stage_f(s0, s1, h2, dis, b2, batch2, meta, wh1a, wh1b, bh1, wh2, bh2):
    return pl.pallas_call(
        _tc_f_body,
        grid=(_NB,),
        in_specs=[
            pl.BlockSpec((_RB, HID), lambda i: (i, 0)),
            pl.BlockSpec((_RB, HID), lambda i: (i, 0)),
            pl.BlockSpec((_RB, HID), lambda i: (i, 0)),
            pl.BlockSpec((_RB, 1), lambda i: (i, 0)),
            pl.BlockSpec((1, HID), lambda i: (0, 0)),
            pl.BlockSpec((_RB, 1), lambda i: (i, 0)),
            pl.BlockSpec((NG, NMETA), lambda i: (0, 0)),
            pl.BlockSpec((HID, HID), lambda i: (0, 0)),
            pl.BlockSpec((NMETA, HID), lambda i: (0, 0)),
            pl.BlockSpec((1, HID), lambda i: (0, 0)),
            pl.BlockSpec((HID, 1), lambda i: (0, 0)),
            pl.BlockSpec((1, 1), lambda i: (0, 0)),
        ],
        out_specs=pl.BlockSpec((NG, 1), lambda i: (0, 0)),
        out_shape=jax.ShapeDtypeStruct((NG, 1), F32),
        scratch_shapes=[
            pltpu.VMEM((NG, HID), F32),
            pltpu.VMEM((NG, HID), F32),
        ],
    )(s0, s1, h2, dis, b2, batch2, meta, wh1a, wh1b, bh1, wh2, bh2)


def kernel(x, edge_index, batch, metadata, W1, b1, W2, b2, Wh1, bh1, Wh2, bh2):
    npad = EPAD - EE
    pad_src = (jnp.arange(npad, dtype=I32) * 131) % NN
    pad_dst = NN + (jnp.arange(npad, dtype=I32) % 16)
    src3 = jnp.concatenate([edge_index[0], pad_src]).reshape(NW, KCH * CH)
    dst3 = jnp.concatenate([edge_index[1], pad_dst]).reshape(NW, KCH * CH)

    zeros_h = jnp.zeros((NSP, HID), F32)
    zeros_d = jnp.zeros((NSP, DW), F32)
    ones_d = jnp.ones((CH, DW), F32)

    deg0, deg1 = _sc_degree(dst3, zeros_d, ones_d)
    h1, u1, dis = _tc_stage_b(x, deg0, deg1, W1)
    s10, s11 = _sc_edge_pass(src3, dst3, u1, zeros_h)
    h2, u2 = _tc_stage_d(s10, s11, h1, dis, b1.reshape(1, HID), W2)
    s20, s21 = _sc_edge_pass(src3, dst3, u2, zeros_h)
    out = _tc_stage_f(s20, s21, h2, dis, b2.reshape(1, HID),
                      batch.reshape(NN, 1), metadata,
                      Wh1[:HID], Wh1[HID:], bh1.reshape(1, HID),
                      Wh2, bh2.reshape(1, 1))
    return out


# speedup vs baseline: 1.1280x; 1.0182x over previous
import functools

import jax
import jax.numpy as jnp
from jax import lax
from jax.experimental import pallas as pl
from jax.experimental.pallas import tpu as pltpu
from jax.experimental.pallas import tpu_sc as plsc

F32 = jnp.float32
I32 = jnp.int32

NN = 10000
EE = 320000
DIN = 128
HID = 64
NG = 64
NMETA = 27

NC, NS = 2, 16
NW = NC * NS
CH = 128
KCH = 80
CHG = 1
NCHK = KCH // CHG
EPAD = NW * KCH * CH
NSP = 10016
RPT = NSP // NS
DW = 8
NBUF = 8

_mesh = plsc.VectorSubcoreMesh(
    core_axis_name="c", subcore_axis_name="s", num_cores=NC, num_subcores=NS)


@functools.partial(
    pl.kernel,
    out_type=(jax.ShapeDtypeStruct((NSP, DW), F32),
              jax.ShapeDtypeStruct((NSP, DW), F32)),
    mesh=_mesh,
    scratch_types=[
        pltpu.VMEM((KCH * CH,), I32),
        pltpu.VMEM((CH, DW), F32),
        pltpu.VMEM_SHARED((NSP, DW), F32),
        pltpu.SemaphoreType.DMA,
    ],
    compiler_params=pltpu.CompilerParams(use_tc_tiling_on_sc=False),
)
def _sc_degree(dst_hbm, zeros_hbm, ones_hbm, deg0_hbm, deg1_hbm,
               dstv, onesv, deg_sh, sem):
    cid = lax.axis_index("c")
    sid = lax.axis_index("s")
    wid = cid * NS + sid
    pltpu.sync_copy(zeros_hbm.at[pl.ds(sid * RPT, RPT)],
                    deg_sh.at[pl.ds(sid * RPT, RPT)])
    pltpu.sync_copy(ones_hbm, onesv)
    pltpu.async_copy(dst_hbm.at[wid], dstv, sem).wait()
    plsc.subcore_barrier()

    def body(j, carry):
        pltpu.async_copy(onesv, deg_sh.at[dstv.at[pl.ds(j * CH, CH)]],
                         sem, add=True)

        @pl.when(j >= 8)
        def _():
            pltpu.make_async_copy(onesv, deg_sh.at[dstv.at[pl.ds(0, CH)]],
                                  sem).wait()

        return carry

    lax.fori_loop(0, KCH, body, 0)
    for _ in range(8):
        pltpu.make_async_copy(onesv, deg_sh.at[dstv.at[pl.ds(0, CH)]],
                              sem).wait()
    plsc.subcore_barrier()

    @pl.when(cid == 0)
    def _():
        pltpu.sync_copy(deg_sh.at[pl.ds(sid * RPT, RPT)],
                        deg0_hbm.at[pl.ds(sid * RPT, RPT)])

    @pl.when(cid == 1)
    def _():
        pltpu.sync_copy(deg_sh.at[pl.ds(sid * RPT, RPT)],
                        deg1_hbm.at[pl.ds(sid * RPT, RPT)])


@functools.partial(
    pl.kernel,
    out_type=(jax.ShapeDtypeStruct((NSP, HID), F32),
              jax.ShapeDtypeStruct((NSP, HID), F32)),
    mesh=_mesh,
    scratch_types=[
        pltpu.VMEM((KCH * CH,), I32),
        pltpu.VMEM((KCH * CH,), I32),
        [pltpu.VMEM((CHG * CH, HID), F32) for _ in range(NBUF)],
        pltpu.VMEM_SHARED((NSP, HID), F32),
        [pltpu.SemaphoreType.DMA for _ in range(NBUF)],
        [pltpu.SemaphoreType.DMA for _ in range(NBUF)],
        pltpu.SemaphoreType.DMA,
    ],
    compiler_params=pltpu.CompilerParams(use_tc_tiling_on_sc=False),
)
def _sc_edge_pass(src_hbm, dst_hbm, u_hbm, zeros_hbm, s0_hbm, s1_hbm,
                  srcv, dstv, bufs, s_sh, gsems, ssems, semi):
    cid = lax.axis_index("c")
    sid = lax.axis_index("s")
    wid = cid * NS + sid
    pltpu.async_copy(src_hbm.at[wid], srcv, semi).wait()
    for b in range(NBUF):
        pltpu.async_copy(u_hbm.at[srcv.at[pl.ds(b * CHG * CH, CHG * CH)]], bufs[b],
                         gsems[b])
    pltpu.async_copy(dst_hbm.at[wid], dstv, semi).wait()
    pltpu.sync_copy(zeros_hbm.at[pl.ds(sid * RPT, RPT)],
                    s_sh.at[pl.ds(sid * RPT, RPT)])
    plsc.subcore_barrier()


    def body(i, carry):
        j0 = NBUF * i
        for b in range(NBUF):
            j = j0 + b

            @pl.when(j < NCHK)
            def _(b=b, j=j):
                pltpu.make_async_copy(
                    u_hbm.at[srcv.at[pl.ds(j * CHG * CH, CHG * CH)]], bufs[b],
                    gsems[b]).wait()
                pltpu.async_copy(bufs[b], s_sh.at[dstv.at[pl.ds(j * CHG * CH, CHG * CH)]],
                                 ssems[b], add=True)

        for b in range(NBUF):
            j = j0 + b

            @pl.when(j + NBUF < NCHK)
            def _(b=b, j=j):
                pltpu.make_async_copy(bufs[b], s_sh.at[dstv.at[pl.ds(0, CHG * CH)]],
                                      ssems[b]).wait()
                pltpu.async_copy(
                    u_hbm.at[srcv.at[pl.ds((j + NBUF) * CHG * CH, CHG * CH)]], bufs[b],
                    gsems[b])

        return carry

    lax.fori_loop(0, (NCHK + NBUF - 1) // NBUF, body, 0)
    for b in range(NBUF):
        pltpu.make_async_copy(bufs[b], s_sh.at[dstv.at[pl.ds(0, CHG * CH)]],
                              ssems[b]).wait()
    plsc.subcore_barrier()

    @pl.when(cid == 0)
    def _():
        pltpu.sync_copy(s_sh.at[pl.ds(sid * RPT, RPT)],
                        s0_hbm.at[pl.ds(sid * RPT, RPT)])

    @pl.when(cid == 1)
    def _():
        pltpu.sync_copy(s_sh.at[pl.ds(sid * RPT, RPT)],
                        s1_hbm.at[pl.ds(sid * RPT, RPT)])


_RB = 400
_NB = NN // _RB


def _tc_b_body(x_ref, d0_ref, d1_ref, w1_ref, h_ref, u_ref, dis_ref):
    deg = d0_ref[:, 0:1] + d1_ref[:, 0:1] + 1.0
    dis = lax.rsqrt(deg)
    h = jnp.dot(x_ref[...], w1_ref[...], preferred_element_type=F32)
    h_ref[...] = h
    u_ref[...] = h * dis
    dis_ref[...] = dis


def _tc_stage_b(x, deg0, deg1, w1):
    return pl.pallas_call(
        _tc_b_body,
        grid=(_NB,),
        in_specs=[
            pl.BlockSpec((_RB, DIN), lambda i: (i, 0)),
            pl.BlockSpec((_RB, DW), lambda i: (i, 0)),
            pl.BlockSpec((_RB, DW), lambda i: (i, 0)),
            pl.BlockSpec((DIN, HID), lambda i: (0, 0)),
        ],
        out_specs=[
            pl.BlockSpec((_RB, HID), lambda i: (i, 0)),
            pl.BlockSpec((_RB, HID), lambda i: (i, 0)),
            pl.BlockSpec((_RB, 1), lambda i: (i, 0)),
        ],
        out_shape=[
            jax.ShapeDtypeStruct((NN, HID), F32),
            jax.ShapeDtypeStruct((NN, HID), F32),
            jax.ShapeDtypeStruct((NN, 1), F32),
        ],
    )(x, deg0, deg1, w1)


def _tc_d_body(s0_ref, s1_ref, h1_ref, dis_ref, b1_ref, w2_ref,
               h2_ref, u2_ref):
    dis = dis_ref[...]
    a = dis * (s0_ref[...] + s1_ref[...]) + dis * dis * h1_ref[...] \
        + b1_ref[...]
    a = jnp.maximum(a, 0.0)
    h2 = jnp.dot(a, w2_ref[...], preferred_element_type=F32)
    h2_ref[...] = h2
    u2_ref[...] = h2 * dis


def _tc_stage_d(s0, s1, h1, dis, b1, w2):
    return pl.pallas_call(
        _tc_d_body,
        grid=(_NB,),
        in_specs=[
            pl.BlockSpec((_RB, HID), lambda i: (i, 0)),
            pl.BlockSpec((_RB, HID), lambda i: (i, 0)),
            pl.BlockSpec((_RB, HID), lambda i: (i, 0)),
            pl.BlockSpec((_RB, 1), lambda i: (i, 0)),
            pl.BlockSpec((1, HID), lambda i: (0, 0)),
            pl.BlockSpec((HID, HID), lambda i: (0, 0)),
        ],
        out_specs=[
            pl.BlockSpec((_RB, HID), lambda i: (i, 0)),
            pl.BlockSpec((_RB, HID), lambda i: (i, 0)),
        ],
        out_shape=[
            jax.ShapeDtypeStruct((NN, HID), F32),
            jax.ShapeDtypeStruct((NN, HID), F32),
        ],
    )(s0, s1, h1, dis, b1, w2)


def _tc_f_body(s0_ref, s1_ref, h2_ref, dis_ref, b2_ref, batch_ref, meta_ref,
               wh1a_ref, wh1b_ref, bh1_ref, wh2_ref, bh2_ref,
               out_ref, acc_e, acc_c):
    i = pl.program_id(0)

    @pl.when(i == 0)
    def _():
        acc_e[...] = jnp.zeros((NG, HID), F32)
        acc_c[...] = jnp.zeros((NG, HID), F32)

    dis = dis_ref[...]
    a = dis * (s0_ref[...] + s1_ref[...]) + dis * dis * h2_ref[...] \
        + b2_ref[...]
    a = jnp.maximum(a, 0.0)
    gids = lax.broadcasted_iota(I32, (_RB, NG), 1)
    onehot = (batch_ref[...] == gids).astype(F32)
    dn = (((0,), (0,)), ((), ()))
    acc_e[...] += lax.dot_general(onehot, a, dn,
                                  preferred_element_type=F32)
    acc_c[...] += lax.dot_general(onehot, jnp.ones((_RB, HID), F32), dn,
                                  preferred_element_type=F32)

    @pl.when(i == _NB - 1)
    def _():
        emb = acc_e[...] / jnp.maximum(acc_c[...], 1.0)
        z = jnp.dot(emb, wh1a_ref[...], preferred_element_type=F32) \
            + jnp.dot(meta_ref[...], wh1b_ref[...],
                      preferred_element_type=F32) + bh1_ref[...]
        z = jnp.maximum(z, 0.0)
        out_ref[...] = jnp.dot(z, wh2_ref[...],
                               preferred_element_type=F32) + bh2_ref[...]


def _tc_stage_f(s0, s1, h2, dis, b2, batch2, meta, wh1a, wh1b, bh1, wh2, bh2):
    return pl.pallas_call(
        _tc_f_body,
        grid=(_NB,),
        in_specs=[
            pl.BlockSpec((_RB, HID), lambda i: (i, 0)),
            pl.BlockSpec((_RB, HID), lambda i: (i, 0)),
            pl.BlockSpec((_RB, HID), lambda i: (i, 0)),
            pl.BlockSpec((_RB, 1), lambda i: (i, 0)),
            pl.BlockSpec((1, HID), lambda i: (0, 0)),
            pl.BlockSpec((_RB, 1), lambda i: (i, 0)),
            pl.BlockSpec((NG, NMETA), lambda i: (0, 0)),
            pl.BlockSpec((HID, HID), lambda i: (0, 0)),
            pl.BlockSpec((NMETA, HID), lambda i: (0, 0)),
            pl.BlockSpec((1, HID), lambda i: (0, 0)),
            pl.BlockSpec((HID, 1), lambda i: (0, 0)),
            pl.BlockSpec((1, 1), lambda i: (0, 0)),
        ],
        out_specs=pl.BlockSpec((NG, 1), lambda i: (0, 0)),
        out_shape=jax.ShapeDtypeStruct((NG, 1), F32),
        scratch_shapes=[
            pltpu.VMEM((NG, HID), F32),
            pltpu.VMEM((NG, HID), F32),
        ],
    )(s0, s1, h2, dis, b2, batch2, meta, wh1a, wh1b, bh1, wh2, bh2)


def kernel(x, edge_index, batch, metadata, W1, b1, W2, b2, Wh1, bh1, Wh2, bh2):
    npad = EPAD - EE
    pad_src = (jnp.arange(npad, dtype=I32) * 131) % NN
    pad_dst = NN + (jnp.arange(npad, dtype=I32) % 16)
    src3 = jnp.concatenate([edge_index[0], pad_src]).reshape(NW, KCH * CH)
    dst3 = jnp.concatenate([edge_index[1], pad_dst]).reshape(NW, KCH * CH)

    zeros_h = jnp.zeros((NSP, HID), F32)
    zeros_d = jnp.zeros((NSP, DW), F32)
    ones_d = jnp.ones((CH, DW), F32)

    deg0, deg1 = _sc_degree(dst3, zeros_d, ones_d)
    h1, u1, dis = _tc_stage_b(x, deg0, deg1, W1)
    s10, s11 = _sc_edge_pass(src3, dst3, u1, zeros_h)
    h2, u2 = _tc_stage_d(s10, s11, h1, dis, b1.reshape(1, HID), W2)
    s20, s21 = _sc_edge_pass(src3, dst3, u2, zeros_h)
    out = _tc_stage_f(s20, s21, h2, dis, b2.reshape(1, HID),
                      batch.reshape(NN, 1), metadata,
                      Wh1[:HID], Wh1[HID:], bh1.reshape(1, HID),
                      Wh2, bh2.reshape(1, 1))
    return out
